# Initial kernel scaffold; baseline (speedup 1.0000x reference)
import jax
import jax.numpy as jnp
from jax.experimental import pallas as pl


def kernel(company_emb, field_emb, nodes, com_id, hier_embed, raw_field_embed, raw_hier_embed, company_table, field_table, W_proj, b_proj, theta, alpha_fields, fc_field_w, fc_field_b, fc_company_w, fc_company_b, w1, b1, w2, b2):
    raise NotImplementedError("write your pallas kernel here")


# speedup vs baseline: 3.4207x; 3.4207x over previous
import jax
import jax.numpy as jnp
from jax import lax
from jax.experimental import pallas as pl
from jax.experimental.pallas import tpu as pltpu
from jax.experimental.pallas import tpu_sc as plsc

NF = 60082
NC = 14695
DIM = 64
B = 8
L = 50
LP = 64
NS = 2 * LP * B
BLK = 2048
NB = (NF + BLK - 1) // BLK
VLEN = NB * BLK
CH = VLEN // 32


def _dense_body(ft, femb, remb, wproj, bproj, fcw, fcwt, fcb, fccw, fccb,
                cemb, ctab, thw, cid, now_v, his_v, w1t, b1, w2t, b2,
                vrow, fewrow, hmwrow, cb16, exm,
                ct_rows, th_g):
    i = pl.program_id(0)

    @pl.when(i == 0)
    def _():
        def gather_c(k, _):
            c = cid[0, k]
            ct_rows[pl.ds(k, 1), :] = ctab[pl.ds(c, 1), :]
            r = thw[pl.ds(c // 128, 1), :]
            lane = lax.broadcasted_iota(jnp.int32, (1, 128), 1)
            th_g[pl.ds(k, 1), :] = jnp.sum(
                jnp.where(lane == c % 128, r, 0.0), axis=1, keepdims=True)
            return 0

        lax.fori_loop(0, B, gather_c, 0)
        th = th_g[...]
        cms = (1.0 - th) * cemb[...] + th * ct_rows[...]
        c8 = (jnp.sum(cms * fccw[...], axis=1, keepdims=True)
              + fccb[0, 0] + fcb[0, 0])
        cb16[...] = jnp.broadcast_to(c8, (B, 16))
        eq = now_v[...][:, :, None] == his_v[...][:, None, :]
        in_his = jnp.any(eq, axis=2)
        in_now = jnp.any(eq, axis=1)
        exm[...] = jnp.concatenate(
            [in_his.astype(jnp.float32), in_now.astype(jnp.float32)],
            axis=1)

    w_eff = jnp.sum(wproj[...] * fcwt[...], axis=0, keepdims=True)
    b_eff = jnp.sum(bproj[...] * fcw[...])
    vrow[...] = (lax.dot_general(w_eff, ft[...],
                                 (((1,), (1,)), ((), ())))
                 + b_eff).reshape(1, 1, BLK)
    fewrow[...] = lax.dot_general(fcw[...], femb[...],
                                  (((1,), (1,)), ((), ()))).reshape(1, 1, BLK)
    h = lax.dot_general(remb[...], w1t[...],
                        (((1,), (0,)), ((), ()))) + b1[...]
    h = jnp.where(h >= 0.0, h, 0.01 * h)
    hm = lax.dot_general(h, w2t[...],
                         (((1,), (0,)), ((), ()))) + b2[...]
    hmwrow[...] = lax.dot_general(fcw[...], hm,
                                  (((1,), (1,)), ((), ()))).reshape(1, 1, BLK)


def _sc_body(vflat, aflat, fewflat, hmwflat, cbvec, sidx, sb, sval_ex, out,
             tbuf, gbuf, outbuf, vbuf, cbuf, idxv, bv, exv, locg, spg):
    cidx = lax.axis_index("c")
    sid = lax.axis_index("s")
    wid = sid * 2 + cidx
    base = wid * CH

    pltpu.sync_copy(vflat.at[pl.ds(base, CH)], vbuf)
    pltpu.sync_copy(cbvec, cbuf)
    pltpu.sync_copy(sidx, idxv)
    pltpu.sync_copy(sb, bv)
    pltpu.sync_copy(sval_ex, exv)

    @pl.when(sid == 0)
    def _():
        pltpu.sync_copy(vflat, tbuf)

    @pl.when(sid == 1)
    def _():
        pltpu.sync_copy(aflat, tbuf)

    @pl.when(sid == 2)
    def _():
        pltpu.sync_copy(fewflat, tbuf)

    @pl.when(sid == 3)
    def _():
        pltpu.sync_copy(hmwflat, tbuf)

    @pl.when(sid < 4)
    def _():
        def g(c, _):
            sl = pl.ds(c * 16, 16)
            gbuf[sl] = plsc.load_gather(tbuf, [idxv[sl]])
            return 0

        lax.fori_loop(0, NS // 16, g, 0)
        pltpu.sync_copy(gbuf, spg.at[pl.ds(sid * NS, NS)])

    for b in range(B):
        cb_b = cbuf[pl.ds(b * 16, 16)]

        def fill(j, _):
            outbuf[pl.ds(b * CH + j * 16, 16)] = vbuf[pl.ds(j * 16, 16)] + cb_b
            return 0

        lax.fori_loop(0, CH // 16, fill, 0)

    plsc.subcore_barrier()

    pltpu.sync_copy(spg, locg)

    def scat(c, _):
        sl = pl.ds(c * 16, 16)
        vg = locg[pl.ds(0 * NS + c * 16, 16)]
        ag = locg[pl.ds(1 * NS + c * 16, 16)]
        fg = locg[pl.ds(2 * NS + c * 16, 16)]
        hg = locg[pl.ds(3 * NS + c * 16, 16)]
        is_now = (c % 8) < 4
        p = jnp.where(is_now, fg, hg)
        s = jnp.where(is_now, hg, fg)
        bvv = bv[sl]
        c8e = plsc.load_gather(cbuf, [bvv * 16])
        val = vg + c8e + ag * (p - vg + exv[sl] * s)
        iv = idxv[sl]
        m = (iv >= base) & (iv < base + CH)
        flat = bvv * CH + (iv - base)
        plsc.store_scatter(outbuf, [flat], val, mask=m)
        return 0

    lax.fori_loop(0, NS // 16, scat, 0)

    for b in range(B):
        pltpu.sync_copy(outbuf.at[pl.ds(b * CH, CH)],
                        out.at[pl.ds(b * VLEN + base, CH)])


def _pad_idx(x):
    return jnp.concatenate(
        [x, jnp.broadcast_to(x[:, -1:], (B, LP - L))], axis=1)


def kernel(company_emb, field_emb, nodes, com_id, hier_embed, raw_field_embed,
           raw_hier_embed, company_table, field_table, W_proj, b_proj, theta,
           alpha_fields, fc_field_w, fc_field_b, fc_company_w, fc_company_b,
           w1, b1, w2, b2):
    nodes = nodes.astype(jnp.int32)
    his = _pad_idx(nodes[:, 0, :])
    now = _pad_idx(nodes[:, 1, :])
    cid2 = com_id.astype(jnp.int32).reshape(1, B)

    thw = jnp.concatenate(
        [theta[:, 0],
         jnp.zeros((115 * 128 - NC,), jnp.float32)]).reshape(115, 128)
    aflat = jnp.concatenate(
        [alpha_fields[:, 0], jnp.zeros((VLEN - NF,), jnp.float32)])
    bproj2 = b_proj.reshape(1, DIM)
    fcb2 = fc_field_b.reshape(1, 1)
    fccb2 = fc_company_b.reshape(1, 1)
    b1r = b1.reshape(1, DIM // 2)
    b2r = b2.reshape(1, DIM)

    blk = lambda r, c: pl.BlockSpec((r, c), lambda i: (0, 0))
    v30, few30, hmw30, cb16, exm = pl.pallas_call(
        _dense_body,
        grid=(NB,),
        in_specs=[
            pl.BlockSpec((BLK, DIM), lambda i: (i, 0)),
            pl.BlockSpec((BLK, DIM), lambda i: (i, 0)),
            pl.BlockSpec((BLK, DIM), lambda i: (i, 0)),
            blk(DIM, DIM),
            blk(1, DIM),
            blk(1, DIM),
            blk(DIM, 1),
            blk(1, 1),
            blk(1, DIM),
            blk(1, 1),
            blk(B, DIM),
            blk(NC, DIM),
            blk(115, 128),
            pl.BlockSpec((1, B), lambda i: (0, 0),
                         memory_space=pltpu.SMEM),
            blk(B, LP),
            blk(B, LP),
            blk(DIM, DIM // 2),
            blk(1, DIM // 2),
            blk(DIM // 2, DIM),
            blk(1, DIM),
        ],
        out_specs=[
            pl.BlockSpec((1, 1, BLK), lambda i: (i, 0, 0)),
            pl.BlockSpec((1, 1, BLK), lambda i: (i, 0, 0)),
            pl.BlockSpec((1, 1, BLK), lambda i: (i, 0, 0)),
            pl.BlockSpec((B, 16), lambda i: (0, 0)),
            pl.BlockSpec((B, 2 * LP), lambda i: (0, 0)),
        ],
        out_shape=[
            jax.ShapeDtypeStruct((NB, 1, BLK), jnp.float32),
            jax.ShapeDtypeStruct((NB, 1, BLK), jnp.float32),
            jax.ShapeDtypeStruct((NB, 1, BLK), jnp.float32),
            jax.ShapeDtypeStruct((B, 16), jnp.float32),
            jax.ShapeDtypeStruct((B, 2 * LP), jnp.float32),
        ],
        scratch_shapes=[
            pltpu.VMEM((B, DIM), jnp.float32),
            pltpu.VMEM((B, 1), jnp.float32),
        ],
    )(field_table, field_emb, raw_field_embed, W_proj, bproj2, fc_field_w,
      fc_field_w.T, fcb2, fc_company_w, fccb2, company_emb, company_table,
      thw, cid2, now, his, w1.T, b1r, w2.T, b2r)

    vflat = v30.reshape(VLEN)
    fewflat = few30.reshape(VLEN)
    hmwflat = hmw30.reshape(VLEN)
    scat_idx = jnp.concatenate([now, his], axis=1).reshape(NS)
    scat_b = jnp.broadcast_to(jnp.arange(B, dtype=jnp.int32)[:, None],
                              (B, 2 * LP)).reshape(NS)
    exflat = exm.reshape(NS)

    mesh = plsc.VectorSubcoreMesh(core_axis_name="c", subcore_axis_name="s")
    out = pl.kernel(
        _sc_body,
        out_type=jax.ShapeDtypeStruct((B * VLEN,), jnp.float32),
        mesh=mesh,
        compiler_params=pltpu.CompilerParams(needs_layout_passes=False),
        scratch_types=[
            pltpu.VMEM((VLEN,), jnp.float32),
            pltpu.VMEM((NS,), jnp.float32),
            pltpu.VMEM((B * CH,), jnp.float32),
            pltpu.VMEM((CH,), jnp.float32),
            pltpu.VMEM((B * 16,), jnp.float32),
            pltpu.VMEM((NS,), jnp.int32),
            pltpu.VMEM((NS,), jnp.int32),
            pltpu.VMEM((NS,), jnp.float32),
            pltpu.VMEM((4 * NS,), jnp.float32),
            pltpu.VMEM_SHARED((4 * NS,), jnp.float32),
        ],
    )(vflat, aflat, fewflat, hmwflat, cb16.reshape(B * 16), scat_idx,
      scat_b, exflat)

    return out.reshape(B, VLEN)[:, :NF]


# speedup vs baseline: 3.4216x; 1.0003x over previous
import jax
import jax.numpy as jnp
from jax import lax
from jax.experimental import pallas as pl
from jax.experimental.pallas import tpu as pltpu
from jax.experimental.pallas import tpu_sc as plsc

NF = 60082
NC = 14695
DIM = 64
B = 8
L = 50
LP = 64
NS = 2 * LP * B
BLK = 2048
NB = (NF + BLK - 1) // BLK
VLEN = NB * BLK
CH = VLEN // 32


def _dense_body(ft, femb, remb, wproj, bproj, fcw, fcwt, fcb, fccw, fccb,
                cemb, ctab, thw, cid, now_v, his_v, w1t, b1, w2t, b2,
                vrow, fewrow, hmwrow, cb16, exm,
                ct_rows, th_g):
    i = pl.program_id(0)

    @pl.when(i == 0)
    def _():
        def gather_c(k, _):
            c = cid[0, k]
            ct_rows[pl.ds(k, 1), :] = ctab[pl.ds(c, 1), :]
            r = thw[pl.ds(c // 128, 1), :]
            lane = lax.broadcasted_iota(jnp.int32, (1, 128), 1)
            th_g[pl.ds(k, 1), :] = jnp.sum(
                jnp.where(lane == c % 128, r, 0.0), axis=1, keepdims=True)
            return 0

        lax.fori_loop(0, B, gather_c, 0)
        th = th_g[...]
        cms = (1.0 - th) * cemb[...] + th * ct_rows[...]
        c8 = (jnp.sum(cms * fccw[...], axis=1, keepdims=True)
              + fccb[0, 0] + fcb[0, 0])
        cb16[...] = jnp.broadcast_to(c8, (B, 16))
        eq = now_v[...][:, :, None] == his_v[...][:, None, :]
        in_his = jnp.any(eq, axis=2)
        in_now = jnp.any(eq, axis=1)
        exm[...] = jnp.concatenate(
            [in_his.astype(jnp.float32), in_now.astype(jnp.float32)],
            axis=1)

    w_eff = jnp.sum(wproj[...] * fcwt[...], axis=0, keepdims=True)
    b_eff = jnp.sum(bproj[...] * fcw[...])
    vrow[...] = (lax.dot_general(w_eff, ft[...],
                                 (((1,), (1,)), ((), ())))
                 + b_eff).reshape(BLK)
    fewrow[...] = lax.dot_general(fcw[...], femb[...],
                                  (((1,), (1,)), ((), ()))).reshape(BLK)
    h = lax.dot_general(remb[...], w1t[...],
                        (((1,), (0,)), ((), ()))) + b1[...]
    h = jnp.where(h >= 0.0, h, 0.01 * h)
    hm = lax.dot_general(h, w2t[...],
                         (((1,), (0,)), ((), ()))) + b2[...]
    hmwrow[...] = lax.dot_general(fcw[...], hm,
                                  (((1,), (1,)), ((), ()))).reshape(BLK)


def _sc_body(vflat, aflat, fewflat, hmwflat, cbvec, sidx, sb, sval_ex, out,
             tbuf, gbuf, outbuf, vbuf, cbuf, idxv, bv, exv, locg, spg):
    cidx = lax.axis_index("c")
    sid = lax.axis_index("s")
    wid = sid * 2 + cidx
    base = wid * CH

    pltpu.sync_copy(vflat.at[pl.ds(base, CH)], vbuf)
    pltpu.sync_copy(cbvec, cbuf)
    pltpu.sync_copy(sidx, idxv)
    pltpu.sync_copy(sb, bv)
    pltpu.sync_copy(sval_ex, exv)

    @pl.when(sid == 0)
    def _():
        pltpu.sync_copy(vflat, tbuf)

    @pl.when(sid == 1)
    def _():
        pltpu.sync_copy(aflat, tbuf)

    @pl.when(sid == 2)
    def _():
        pltpu.sync_copy(fewflat, tbuf)

    @pl.when(sid == 3)
    def _():
        pltpu.sync_copy(hmwflat, tbuf)

    @pl.when(sid < 4)
    def _():
        def g(c, _):
            sl = pl.ds(c * 16, 16)
            gbuf[sl] = plsc.load_gather(tbuf, [idxv[sl]])
            return 0

        lax.fori_loop(0, NS // 16, g, 0)
        pltpu.sync_copy(gbuf, spg.at[pl.ds(sid * NS, NS)])

    for b in range(B):
        cb_b = cbuf[pl.ds(b * 16, 16)]

        def fill(j, _):
            outbuf[pl.ds(b * CH + j * 16, 16)] = vbuf[pl.ds(j * 16, 16)] + cb_b
            return 0

        lax.fori_loop(0, CH // 16, fill, 0)

    plsc.subcore_barrier()

    pltpu.sync_copy(spg, locg)

    def scat(c, _):
        sl = pl.ds(c * 16, 16)
        vg = locg[pl.ds(0 * NS + c * 16, 16)]
        ag = locg[pl.ds(1 * NS + c * 16, 16)]
        fg = locg[pl.ds(2 * NS + c * 16, 16)]
        hg = locg[pl.ds(3 * NS + c * 16, 16)]
        is_now = (c % 8) < 4
        p = jnp.where(is_now, fg, hg)
        s = jnp.where(is_now, hg, fg)
        bvv = bv[sl]
        c8e = plsc.load_gather(cbuf, [bvv * 16])
        val = vg + c8e + ag * (p - vg + exv[sl] * s)
        iv = idxv[sl]
        m = (iv >= base) & (iv < base + CH)
        flat = bvv * CH + (iv - base)
        plsc.store_scatter(outbuf, [flat], val, mask=m)
        return 0

    lax.fori_loop(0, NS // 16, scat, 0)

    for b in range(B):
        pltpu.sync_copy(outbuf.at[pl.ds(b * CH, CH)],
                        out.at[pl.ds(b * VLEN + base, CH)])


def _pad_idx(x):
    return jnp.concatenate(
        [x, jnp.broadcast_to(x[:, -1:], (B, LP - L))], axis=1)


def kernel(company_emb, field_emb, nodes, com_id, hier_embed, raw_field_embed,
           raw_hier_embed, company_table, field_table, W_proj, b_proj, theta,
           alpha_fields, fc_field_w, fc_field_b, fc_company_w, fc_company_b,
           w1, b1, w2, b2):
    nodes = nodes.astype(jnp.int32)
    his = _pad_idx(nodes[:, 0, :])
    now = _pad_idx(nodes[:, 1, :])
    cid2 = com_id.astype(jnp.int32).reshape(1, B)

    thw = jnp.concatenate(
        [theta[:, 0],
         jnp.zeros((115 * 128 - NC,), jnp.float32)]).reshape(115, 128)
    aflat = jnp.concatenate(
        [alpha_fields[:, 0], jnp.zeros((VLEN - NF,), jnp.float32)])
    bproj2 = b_proj.reshape(1, DIM)
    fcb2 = fc_field_b.reshape(1, 1)
    fccb2 = fc_company_b.reshape(1, 1)
    b1r = b1.reshape(1, DIM // 2)
    b2r = b2.reshape(1, DIM)

    blk = lambda r, c: pl.BlockSpec((r, c), lambda i: (0, 0))
    vflat, fewflat, hmwflat, cb16, exm = pl.pallas_call(
        _dense_body,
        grid=(NB,),
        in_specs=[
            pl.BlockSpec((BLK, DIM), lambda i: (i, 0)),
            pl.BlockSpec((BLK, DIM), lambda i: (i, 0)),
            pl.BlockSpec((BLK, DIM), lambda i: (i, 0)),
            blk(DIM, DIM),
            blk(1, DIM),
            blk(1, DIM),
            blk(DIM, 1),
            blk(1, 1),
            blk(1, DIM),
            blk(1, 1),
            blk(B, DIM),
            blk(NC, DIM),
            blk(115, 128),
            pl.BlockSpec((1, B), lambda i: (0, 0),
                         memory_space=pltpu.SMEM),
            blk(B, LP),
            blk(B, LP),
            blk(DIM, DIM // 2),
            blk(1, DIM // 2),
            blk(DIM // 2, DIM),
            blk(1, DIM),
        ],
        out_specs=[
            pl.BlockSpec((BLK,), lambda i: (i,)),
            pl.BlockSpec((BLK,), lambda i: (i,)),
            pl.BlockSpec((BLK,), lambda i: (i,)),
            pl.BlockSpec((B, 16), lambda i: (0, 0)),
            pl.BlockSpec((B, 2 * LP), lambda i: (0, 0)),
        ],
        out_shape=[
            jax.ShapeDtypeStruct((VLEN,), jnp.float32),
            jax.ShapeDtypeStruct((VLEN,), jnp.float32),
            jax.ShapeDtypeStruct((VLEN,), jnp.float32),
            jax.ShapeDtypeStruct((B, 16), jnp.float32),
            jax.ShapeDtypeStruct((B, 2 * LP), jnp.float32),
        ],
        scratch_shapes=[
            pltpu.VMEM((B, DIM), jnp.float32),
            pltpu.VMEM((B, 1), jnp.float32),
        ],
    )(field_table, field_emb, raw_field_embed, W_proj, bproj2, fc_field_w,
      fc_field_w.T, fcb2, fc_company_w, fccb2, company_emb, company_table,
      thw, cid2, now, his, w1.T, b1r, w2.T, b2r)

    scat_idx = jnp.concatenate([now, his], axis=1).reshape(NS)
    scat_b = jnp.broadcast_to(jnp.arange(B, dtype=jnp.int32)[:, None],
                              (B, 2 * LP)).reshape(NS)
    exflat = exm.reshape(NS)

    mesh = plsc.VectorSubcoreMesh(core_axis_name="c", subcore_axis_name="s")
    out = pl.kernel(
        _sc_body,
        out_type=jax.ShapeDtypeStruct((B * VLEN,), jnp.float32),
        mesh=mesh,
        compiler_params=pltpu.CompilerParams(needs_layout_passes=False),
        scratch_types=[
            pltpu.VMEM((VLEN,), jnp.float32),
            pltpu.VMEM((NS,), jnp.float32),
            pltpu.VMEM((B * CH,), jnp.float32),
            pltpu.VMEM((CH,), jnp.float32),
            pltpu.VMEM((B * 16,), jnp.float32),
            pltpu.VMEM((NS,), jnp.int32),
            pltpu.VMEM((NS,), jnp.int32),
            pltpu.VMEM((NS,), jnp.float32),
            pltpu.VMEM((4 * NS,), jnp.float32),
            pltpu.VMEM_SHARED((4 * NS,), jnp.float32),
        ],
    )(vflat, aflat, fewflat, hmwflat, cb16.reshape(B * 16), scat_idx,
      scat_b, exflat)

    return out.reshape(B, VLEN)[:, :NF]


# speedup vs baseline: 3.8061x; 1.1124x over previous
import jax
import jax.numpy as jnp
from jax import lax
from jax.experimental import pallas as pl
from jax.experimental.pallas import tpu as pltpu
from jax.experimental.pallas import tpu_sc as plsc

NF = 60082
NC = 14695
DIM = 64
B = 8
L = 50
LP = 64
NS = 2 * LP * B
BLK = 4096
NB = (NF + BLK - 1) // BLK
VLEN = NB * BLK
CH = VLEN // 32


def _dense_body(ft, femb, remb, wproj, bproj, fcw, fcwt, fcb, fccw, fccb,
                cemb, ctab, thw, cid, now_v, his_v, w1t, b1, w2t, b2,
                vrow, fewrow, hmwrow, cb16, exm,
                ct_rows, th_g):
    i = pl.program_id(0)

    @pl.when(i == 0)
    def _():
        def gather_c(k, _):
            c = cid[0, k]
            ct_rows[pl.ds(k, 1), :] = ctab[pl.ds(c, 1), :]
            r = thw[pl.ds(c // 128, 1), :]
            lane = lax.broadcasted_iota(jnp.int32, (1, 128), 1)
            th_g[pl.ds(k, 1), :] = jnp.sum(
                jnp.where(lane == c % 128, r, 0.0), axis=1, keepdims=True)
            return 0

        lax.fori_loop(0, B, gather_c, 0)
        th = th_g[...]
        cms = (1.0 - th) * cemb[...] + th * ct_rows[...]
        c8 = (jnp.sum(cms * fccw[...], axis=1, keepdims=True)
              + fccb[0, 0] + fcb[0, 0])
        cb16[...] = jnp.broadcast_to(c8, (B, 16))
        eq = now_v[...][:, :, None] == his_v[...][:, None, :]
        in_his = jnp.any(eq, axis=2)
        in_now = jnp.any(eq, axis=1)
        exm[...] = jnp.concatenate(
            [in_his.astype(jnp.float32), in_now.astype(jnp.float32)],
            axis=1)

    w_eff = jnp.sum(wproj[...] * fcwt[...], axis=0, keepdims=True)
    b_eff = jnp.sum(bproj[...] * fcw[...])
    vrow[...] = (lax.dot_general(w_eff, ft[...],
                                 (((1,), (1,)), ((), ())))
                 + b_eff).reshape(BLK)
    fewrow[...] = lax.dot_general(fcw[...], femb[...],
                                  (((1,), (1,)), ((), ()))).reshape(BLK)
    h = lax.dot_general(remb[...], w1t[...],
                        (((1,), (0,)), ((), ()))) + b1[...]
    h = jnp.maximum(h, 0.01 * h)
    hm = lax.dot_general(h, w2t[...],
                         (((1,), (0,)), ((), ()))) + b2[...]
    hmwrow[...] = lax.dot_general(fcw[...], hm,
                                  (((1,), (1,)), ((), ()))).reshape(BLK)


def _sc_body(vflat, aflat, fewflat, hmwflat, cbvec, sidx, sb, sval_ex, out,
             tbuf, gbuf, outbuf, vbuf, cbuf, idxv, bv, exv, locg, spg,
             sem, sem2):
    cidx = lax.axis_index("c")
    sid = lax.axis_index("s")
    wid = sid * 2 + cidx
    base = wid * CH

    h1 = pltpu.async_copy(vflat.at[pl.ds(base, CH)], vbuf, sem)
    h2 = pltpu.async_copy(cbvec, cbuf, sem)
    h3 = pltpu.async_copy(sidx, idxv, sem)
    h4 = pltpu.async_copy(sb, bv, sem)
    h5 = pltpu.async_copy(sval_ex, exv, sem)

    @pl.when(sid == 0)
    def _():
        pltpu.async_copy(vflat, tbuf, sem2).wait()

    @pl.when(sid == 1)
    def _():
        pltpu.async_copy(aflat, tbuf, sem2).wait()

    @pl.when(sid == 2)
    def _():
        pltpu.async_copy(fewflat, tbuf, sem2).wait()

    @pl.when(sid == 3)
    def _():
        pltpu.async_copy(hmwflat, tbuf, sem2).wait()

    h1.wait()
    h2.wait()
    h3.wait()
    h4.wait()
    h5.wait()

    @pl.when(sid < 4)
    def _():
        def g(c, _):
            sl = pl.ds(c * 16, 16)
            gbuf[sl] = plsc.load_gather(tbuf, [idxv[sl]])
            return 0

        lax.fori_loop(0, NS // 16, g, 0)
        pltpu.sync_copy(gbuf, spg.at[pl.ds(sid * NS, NS)])

    cbs = [cbuf[pl.ds(b * 16, 16)] for b in range(B)]

    def fill(j, _):
        vv = vbuf[pl.ds(j * 16, 16)]
        for b in range(B):
            outbuf[pl.ds(b * CH + j * 16, 16)] = vv + cbs[b]
        return 0

    lax.fori_loop(0, CH // 16, fill, 0)

    plsc.subcore_barrier()

    pltpu.sync_copy(spg, locg)

    def scat(c, _):
        sl = pl.ds(c * 16, 16)
        vg = locg[pl.ds(0 * NS + c * 16, 16)]
        ag = locg[pl.ds(1 * NS + c * 16, 16)]
        fg = locg[pl.ds(2 * NS + c * 16, 16)]
        hg = locg[pl.ds(3 * NS + c * 16, 16)]
        is_now = (c % 8) < 4
        p = jnp.where(is_now, fg, hg)
        s = jnp.where(is_now, hg, fg)
        bvv = bv[sl]
        c8e = plsc.load_gather(cbuf, [bvv * 16])
        val = vg + c8e + ag * (p - vg + exv[sl] * s)
        iv = idxv[sl]
        m = (iv >= base) & (iv < base + CH)
        flat = bvv * CH + (iv - base)
        plsc.store_scatter(outbuf, [flat], val, mask=m)
        return 0

    lax.fori_loop(0, NS // 16, scat, 0)

    hs = [pltpu.async_copy(outbuf.at[pl.ds(b * CH, CH)],
                           out.at[pl.ds(b * VLEN + base, CH)], sem)
          for b in range(B)]
    for h in hs:
        h.wait()


def _pad_idx(x):
    return jnp.concatenate(
        [x, jnp.broadcast_to(x[:, -1:], (B, LP - L))], axis=1)


def kernel(company_emb, field_emb, nodes, com_id, hier_embed, raw_field_embed,
           raw_hier_embed, company_table, field_table, W_proj, b_proj, theta,
           alpha_fields, fc_field_w, fc_field_b, fc_company_w, fc_company_b,
           w1, b1, w2, b2):
    nodes = nodes.astype(jnp.int32)
    his = _pad_idx(nodes[:, 0, :])
    now = _pad_idx(nodes[:, 1, :])
    cid2 = com_id.astype(jnp.int32).reshape(1, B)

    thw = jnp.concatenate(
        [theta[:, 0],
         jnp.zeros((115 * 128 - NC,), jnp.float32)]).reshape(115, 128)
    aflat = jnp.concatenate(
        [alpha_fields[:, 0], jnp.zeros((VLEN - NF,), jnp.float32)])
    bproj2 = b_proj.reshape(1, DIM)
    fcb2 = fc_field_b.reshape(1, 1)
    fccb2 = fc_company_b.reshape(1, 1)
    b1r = b1.reshape(1, DIM // 2)
    b2r = b2.reshape(1, DIM)

    blk = lambda r, c: pl.BlockSpec((r, c), lambda i: (0, 0))
    vflat, fewflat, hmwflat, cb16, exm = pl.pallas_call(
        _dense_body,
        grid=(NB,),
        in_specs=[
            pl.BlockSpec((BLK, DIM), lambda i: (i, 0)),
            pl.BlockSpec((BLK, DIM), lambda i: (i, 0)),
            pl.BlockSpec((BLK, DIM), lambda i: (i, 0)),
            blk(DIM, DIM),
            blk(1, DIM),
            blk(1, DIM),
            blk(DIM, 1),
            blk(1, 1),
            blk(1, DIM),
            blk(1, 1),
            blk(B, DIM),
            blk(NC, DIM),
            blk(115, 128),
            pl.BlockSpec((1, B), lambda i: (0, 0),
                         memory_space=pltpu.SMEM),
            blk(B, LP),
            blk(B, LP),
            blk(DIM, DIM // 2),
            blk(1, DIM // 2),
            blk(DIM // 2, DIM),
            blk(1, DIM),
        ],
        out_specs=[
            pl.BlockSpec((BLK,), lambda i: (i,)),
            pl.BlockSpec((BLK,), lambda i: (i,)),
            pl.BlockSpec((BLK,), lambda i: (i,)),
            pl.BlockSpec((B, 16), lambda i: (0, 0)),
            pl.BlockSpec((B, 2 * LP), lambda i: (0, 0)),
        ],
        out_shape=[
            jax.ShapeDtypeStruct((VLEN,), jnp.float32),
            jax.ShapeDtypeStruct((VLEN,), jnp.float32),
            jax.ShapeDtypeStruct((VLEN,), jnp.float32),
            jax.ShapeDtypeStruct((B, 16), jnp.float32),
            jax.ShapeDtypeStruct((B, 2 * LP), jnp.float32),
        ],
        scratch_shapes=[
            pltpu.VMEM((B, DIM), jnp.float32),
            pltpu.VMEM((B, 1), jnp.float32),
        ],
    )(field_table, field_emb, raw_field_embed, W_proj, bproj2, fc_field_w,
      fc_field_w.T, fcb2, fc_company_w, fccb2, company_emb, company_table,
      thw, cid2, now, his, w1.T, b1r, w2.T, b2r)

    scat_idx = jnp.concatenate([now, his], axis=1).reshape(NS)
    scat_b = jnp.broadcast_to(jnp.arange(B, dtype=jnp.int32)[:, None],
                              (B, 2 * LP)).reshape(NS)
    exflat = exm.reshape(NS)

    mesh = plsc.VectorSubcoreMesh(core_axis_name="c", subcore_axis_name="s")
    out = pl.kernel(
        _sc_body,
        out_type=jax.ShapeDtypeStruct((B * VLEN,), jnp.float32),
        mesh=mesh,
        compiler_params=pltpu.CompilerParams(needs_layout_passes=False),
        scratch_types=[
            pltpu.VMEM((VLEN,), jnp.float32),
            pltpu.VMEM((NS,), jnp.float32),
            pltpu.VMEM((B * CH,), jnp.float32),
            pltpu.VMEM((CH,), jnp.float32),
            pltpu.VMEM((B * 16,), jnp.float32),
            pltpu.VMEM((NS,), jnp.int32),
            pltpu.VMEM((NS,), jnp.int32),
            pltpu.VMEM((NS,), jnp.float32),
            pltpu.VMEM((4 * NS,), jnp.float32),
            pltpu.VMEM_SHARED((4 * NS,), jnp.float32),
            pltpu.SemaphoreType.DMA,
            pltpu.SemaphoreType.DMA,
        ],
    )(vflat, aflat, fewflat, hmwflat, cb16.reshape(B * 16), scat_idx,
      scat_b, exflat)

    return out.reshape(B, VLEN)[:, :NF]


# speedup vs baseline: 3.8148x; 1.0023x over previous
import jax
import jax.numpy as jnp
from jax import lax
from jax.experimental import pallas as pl
from jax.experimental.pallas import tpu as pltpu
from jax.experimental.pallas import tpu_sc as plsc

NF = 60082
NC = 14695
DIM = 64
B = 8
L = 50
LP = 64
NS = 2 * LP * B
BLK = 4096
NB = (NF + BLK - 1) // BLK
VLEN = NB * BLK
CH = VLEN // 32


def _dense_body(ft, femb, remb, wproj, bproj, fcw, fcwt, fcb, fccw, fccb,
                cemb, ctab, thw, cid, now_v, his_v, w1t, b1, w2t, b2,
                vrow, fewrow, hmwrow, cb16, exm,
                ct_rows, th_g):
    i = pl.program_id(0)

    @pl.when(i == 0)
    def _():
        def gather_c(k, _):
            c = cid[0, k]
            ct_rows[pl.ds(k, 1), :] = ctab[pl.ds(c, 1), :]
            r = thw[pl.ds(c // 128, 1), :]
            lane = lax.broadcasted_iota(jnp.int32, (1, 128), 1)
            th_g[pl.ds(k, 1), :] = jnp.sum(
                jnp.where(lane == c % 128, r, 0.0), axis=1, keepdims=True)
            return 0

        lax.fori_loop(0, B, gather_c, 0)
        th = th_g[...]
        cms = (1.0 - th) * cemb[...] + th * ct_rows[...]
        c8 = (jnp.sum(cms * fccw[...], axis=1, keepdims=True)
              + fccb[0, 0] + fcb[0, 0])
        cb16[...] = jnp.broadcast_to(c8, (B, 16))
        eq = now_v[...][:, :, None] == his_v[...][:, None, :]
        in_his = jnp.any(eq, axis=2)
        in_now = jnp.any(eq, axis=1)
        exm[...] = jnp.concatenate(
            [in_his.astype(jnp.float32), in_now.astype(jnp.float32)],
            axis=1)

    w_eff = jnp.sum(wproj[...] * fcwt[...], axis=0, keepdims=True)
    b_eff = jnp.sum(bproj[...] * fcw[...])
    vrow[...] = (lax.dot_general(w_eff, ft[...],
                                 (((1,), (1,)), ((), ())))
                 + b_eff).reshape(BLK)
    fewrow[...] = lax.dot_general(fcw[...], femb[...],
                                  (((1,), (1,)), ((), ()))).reshape(BLK)
    h = lax.dot_general(remb[...], w1t[...],
                        (((1,), (0,)), ((), ()))) + b1[...]
    h = jnp.maximum(h, 0.01 * h)
    hm = lax.dot_general(h, w2t[...],
                         (((1,), (0,)), ((), ()))) + b2[...]
    hmwrow[...] = lax.dot_general(fcw[...], hm,
                                  (((1,), (1,)), ((), ()))).reshape(BLK)


def _sc_body(vflat, aflat, fewflat, hmwflat, cbvec, sidx, sb, sval_ex, out,
             tbuf, gbuf, outbuf, vbuf, cbuf, idxv, bv, exv, locg, spg,
             sem, sem2):
    cidx = lax.axis_index("c")
    sid = lax.axis_index("s")
    wid = sid * 2 + cidx
    base = wid * CH

    h1 = pltpu.async_copy(vflat.at[pl.ds(base, CH)], vbuf, sem)
    h2 = pltpu.async_copy(cbvec, cbuf, sem)
    h3 = pltpu.async_copy(sidx, idxv, sem)
    h4 = pltpu.async_copy(sb, bv, sem)
    h5 = pltpu.async_copy(sval_ex, exv, sem)

    @pl.when(sid == 0)
    def _():
        pltpu.async_copy(vflat, tbuf, sem2).wait()

    @pl.when(sid == 1)
    def _():
        pltpu.async_copy(aflat, tbuf, sem2).wait()

    @pl.when(sid == 2)
    def _():
        pltpu.async_copy(fewflat, tbuf, sem2).wait()

    @pl.when(sid == 3)
    def _():
        pltpu.async_copy(hmwflat, tbuf, sem2).wait()

    h1.wait()
    h2.wait()
    h3.wait()
    h4.wait()
    h5.wait()

    @pl.when(sid < 4)
    def _():
        def g(c, _):
            sl = pl.ds(c * 16, 16)
            gbuf[sl] = plsc.load_gather(tbuf, [idxv[sl]])
            return 0

        lax.fori_loop(0, NS // 16, g, 0)
        pltpu.sync_copy(gbuf, spg.at[pl.ds(sid * NS, NS)])

    cbs = [cbuf[pl.ds(b * 16, 16)] for b in range(B)]

    def fill(j, _):
        vv = vbuf[pl.ds(j * 16, 16)]
        for b in range(B):
            outbuf[pl.ds(b * CH + j * 16, 16)] = vv + cbs[b]
        return 0

    lax.fori_loop(0, CH // 16, fill, 0)

    plsc.subcore_barrier()

    pltpu.sync_copy(spg, locg)

    def scat(c, _):
        sl = pl.ds(c * 16, 16)
        vg = locg[pl.ds(0 * NS + c * 16, 16)]
        ag = locg[pl.ds(1 * NS + c * 16, 16)]
        fg = locg[pl.ds(2 * NS + c * 16, 16)]
        hg = locg[pl.ds(3 * NS + c * 16, 16)]
        is_now = (c % 8) < 4
        p = jnp.where(is_now, fg, hg)
        s = jnp.where(is_now, hg, fg)
        bvv = bv[sl]
        c8e = plsc.load_gather(cbuf, [bvv * 16])
        val = vg + c8e + ag * (p - vg + exv[sl] * s)
        iv = idxv[sl]
        m = (iv >= base) & (iv < base + CH)
        flat = bvv * CH + (iv - base)
        plsc.store_scatter(outbuf, [flat], val, mask=m)
        return 0

    lax.fori_loop(0, NS // 16, scat, 0)

    hs = [pltpu.async_copy(outbuf.at[pl.ds(b * CH, CH)],
                           out.at[pl.ds(b * VLEN + base, CH)], sem)
          for b in range(B)]
    for h in hs:
        h.wait()


def _pad_idx(x):
    return jnp.concatenate(
        [x, jnp.broadcast_to(x[:, -1:], (B, LP - L))], axis=1)


def kernel(company_emb, field_emb, nodes, com_id, hier_embed, raw_field_embed,
           raw_hier_embed, company_table, field_table, W_proj, b_proj, theta,
           alpha_fields, fc_field_w, fc_field_b, fc_company_w, fc_company_b,
           w1, b1, w2, b2):
    nodes = nodes.astype(jnp.int32)
    his = _pad_idx(nodes[:, 0, :])
    now = _pad_idx(nodes[:, 1, :])
    cid2 = com_id.astype(jnp.int32).reshape(1, B)

    thw = jnp.concatenate(
        [theta[:, 0],
         jnp.zeros((115 * 128 - NC,), jnp.float32)]).reshape(115, 128)
    aflat = jnp.concatenate(
        [alpha_fields[:, 0], jnp.zeros((VLEN - NF,), jnp.float32)])
    bproj2 = b_proj.reshape(1, DIM)
    fcb2 = fc_field_b.reshape(1, 1)
    fccb2 = fc_company_b.reshape(1, 1)
    b1r = b1.reshape(1, DIM // 2)
    b2r = b2.reshape(1, DIM)

    blk = lambda r, c: pl.BlockSpec((r, c), lambda i: (0, 0))
    vflat, fewflat, hmwflat, cb16, exm = pl.pallas_call(
        _dense_body,
        grid=(NB,),
        compiler_params=pltpu.CompilerParams(
            allow_input_fusion=[True, True, True] + [False] * 17),
        in_specs=[
            pl.BlockSpec((BLK, DIM), lambda i: (i, 0)),
            pl.BlockSpec((BLK, DIM), lambda i: (i, 0)),
            pl.BlockSpec((BLK, DIM), lambda i: (i, 0)),
            blk(DIM, DIM),
            blk(1, DIM),
            blk(1, DIM),
            blk(DIM, 1),
            blk(1, 1),
            blk(1, DIM),
            blk(1, 1),
            blk(B, DIM),
            blk(NC, DIM),
            blk(115, 128),
            pl.BlockSpec((1, B), lambda i: (0, 0),
                         memory_space=pltpu.SMEM),
            blk(B, LP),
            blk(B, LP),
            blk(DIM, DIM // 2),
            blk(1, DIM // 2),
            blk(DIM // 2, DIM),
            blk(1, DIM),
        ],
        out_specs=[
            pl.BlockSpec((BLK,), lambda i: (i,)),
            pl.BlockSpec((BLK,), lambda i: (i,)),
            pl.BlockSpec((BLK,), lambda i: (i,)),
            pl.BlockSpec((B, 16), lambda i: (0, 0)),
            pl.BlockSpec((B, 2 * LP), lambda i: (0, 0)),
        ],
        out_shape=[
            jax.ShapeDtypeStruct((VLEN,), jnp.float32),
            jax.ShapeDtypeStruct((VLEN,), jnp.float32),
            jax.ShapeDtypeStruct((VLEN,), jnp.float32),
            jax.ShapeDtypeStruct((B, 16), jnp.float32),
            jax.ShapeDtypeStruct((B, 2 * LP), jnp.float32),
        ],
        scratch_shapes=[
            pltpu.VMEM((B, DIM), jnp.float32),
            pltpu.VMEM((B, 1), jnp.float32),
        ],
    )(field_table, field_emb, raw_field_embed, W_proj, bproj2, fc_field_w,
      fc_field_w.T, fcb2, fc_company_w, fccb2, company_emb, company_table,
      thw, cid2, now, his, w1.T, b1r, w2.T, b2r)

    scat_idx = jnp.concatenate([now, his], axis=1).reshape(NS)
    scat_b = jnp.broadcast_to(jnp.arange(B, dtype=jnp.int32)[:, None],
                              (B, 2 * LP)).reshape(NS)
    exflat = exm.reshape(NS)

    mesh = plsc.VectorSubcoreMesh(core_axis_name="c", subcore_axis_name="s")
    out = pl.kernel(
        _sc_body,
        out_type=jax.ShapeDtypeStruct((B * VLEN,), jnp.float32),
        mesh=mesh,
        compiler_params=pltpu.CompilerParams(needs_layout_passes=False),
        scratch_types=[
            pltpu.VMEM((VLEN,), jnp.float32),
            pltpu.VMEM((NS,), jnp.float32),
            pltpu.VMEM((B * CH,), jnp.float32),
            pltpu.VMEM((CH,), jnp.float32),
            pltpu.VMEM((B * 16,), jnp.float32),
            pltpu.VMEM((NS,), jnp.int32),
            pltpu.VMEM((NS,), jnp.int32),
            pltpu.VMEM((NS,), jnp.float32),
            pltpu.VMEM((4 * NS,), jnp.float32),
            pltpu.VMEM_SHARED((4 * NS,), jnp.float32),
            pltpu.SemaphoreType.DMA,
            pltpu.SemaphoreType.DMA,
        ],
    )(vflat, aflat, fewflat, hmwflat, cb16.reshape(B * 16), scat_idx,
      scat_b, exflat)

    return out.reshape(B, VLEN)[:, :NF]


# speedup vs baseline: 3.8163x; 1.0004x over previous
import jax
import jax.numpy as jnp
from jax import lax
from jax.experimental import pallas as pl
from jax.experimental.pallas import tpu as pltpu
from jax.experimental.pallas import tpu_sc as plsc

NF = 60082
NC = 14695
DIM = 64
B = 8
L = 50
LP = 64
NS = 2 * LP * B
BLK = 4096
NB = (NF + BLK - 1) // BLK
VLEN = NB * BLK
CH = VLEN // 32


def _dense_body(ft, femb, remb, wproj, bproj, fcw, fcwt, fcb, fccw, fccb,
                cemb, ctab, thw, cid, now_v, his_v, w1t, b1, w2t, b2,
                vrow, fewrow, hmwrow, cb16, exm,
                ct_rows, th_g):
    i = pl.program_id(0)

    @pl.when(i == 0)
    def _():
        def gather_c(k, _):
            c = cid[0, k]
            ct_rows[pl.ds(k, 1), :] = ctab[pl.ds(c, 1), :]
            r = thw[pl.ds(c // 128, 1), :]
            lane = lax.broadcasted_iota(jnp.int32, (1, 128), 1)
            th_g[pl.ds(k, 1), :] = jnp.sum(
                jnp.where(lane == c % 128, r, 0.0), axis=1, keepdims=True)
            return 0

        lax.fori_loop(0, B, gather_c, 0)
        th = th_g[...]
        cms = (1.0 - th) * cemb[...] + th * ct_rows[...]
        c8 = (jnp.sum(cms * fccw[...], axis=1, keepdims=True)
              + fccb[0, 0] + fcb[0, 0])
        cb16[...] = jnp.broadcast_to(c8, (B, 16))
        eq = now_v[...][:, :, None] == his_v[...][:, None, :]
        in_his = jnp.any(eq, axis=2)
        in_now = jnp.any(eq, axis=1)
        exm[...] = jnp.concatenate(
            [in_his.astype(jnp.float32), in_now.astype(jnp.float32)],
            axis=1)

    bf = jnp.bfloat16
    f32 = jnp.float32
    w_eff = jnp.sum(wproj[...] * fcwt[...], axis=0, keepdims=True)
    b_eff = jnp.sum(bproj[...] * fcw[...])
    fcw_b = fcw[...].astype(bf)
    vrow[...] = (lax.dot_general(w_eff.astype(bf), ft[...],
                                 (((1,), (1,)), ((), ())),
                                 preferred_element_type=f32)
                 + b_eff).reshape(BLK)
    fewrow[...] = lax.dot_general(fcw_b, femb[...],
                                  (((1,), (1,)), ((), ())),
                                  preferred_element_type=f32).reshape(BLK)
    h = lax.dot_general(remb[...], w1t[...].astype(bf),
                        (((1,), (0,)), ((), ())),
                        preferred_element_type=f32) + b1[...]
    h = jnp.maximum(h, 0.01 * h)
    hm = lax.dot_general(h.astype(bf), w2t[...].astype(bf),
                         (((1,), (0,)), ((), ())),
                         preferred_element_type=f32) + b2[...]
    hmwrow[...] = lax.dot_general(fcw_b, hm.astype(bf),
                                  (((1,), (1,)), ((), ())),
                                  preferred_element_type=f32).reshape(BLK)


def _sc_body(vflat, aflat, fewflat, hmwflat, cbvec, sidx, sb, sval_ex, out,
             tbuf, gbuf, outbuf, vbuf, cbuf, idxv, bv, exv, locg, spg,
             sem, sem2):
    cidx = lax.axis_index("c")
    sid = lax.axis_index("s")
    wid = sid * 2 + cidx
    base = wid * CH

    h1 = pltpu.async_copy(vflat.at[pl.ds(base, CH)], vbuf, sem)
    h2 = pltpu.async_copy(cbvec, cbuf, sem)
    h3 = pltpu.async_copy(sidx, idxv, sem)
    h4 = pltpu.async_copy(sb, bv, sem)
    h5 = pltpu.async_copy(sval_ex, exv, sem)

    @pl.when(sid == 0)
    def _():
        pltpu.async_copy(vflat, tbuf, sem2).wait()

    @pl.when(sid == 1)
    def _():
        pltpu.async_copy(aflat, tbuf, sem2).wait()

    @pl.when(sid == 2)
    def _():
        pltpu.async_copy(fewflat, tbuf, sem2).wait()

    @pl.when(sid == 3)
    def _():
        pltpu.async_copy(hmwflat, tbuf, sem2).wait()

    h1.wait()
    h2.wait()
    h3.wait()
    h4.wait()
    h5.wait()

    @pl.when(sid < 4)
    def _():
        def g(c, _):
            sl = pl.ds(c * 16, 16)
            gbuf[sl] = plsc.load_gather(tbuf, [idxv[sl]])
            return 0

        lax.fori_loop(0, NS // 16, g, 0)
        pltpu.sync_copy(gbuf, spg.at[pl.ds(sid * NS, NS)])

    cbs = [cbuf[pl.ds(b * 16, 16)] for b in range(B)]

    def fill(j, _):
        vv = vbuf[pl.ds(j * 16, 16)]
        for b in range(B):
            outbuf[pl.ds(b * CH + j * 16, 16)] = vv + cbs[b]
        return 0

    lax.fori_loop(0, CH // 16, fill, 0)

    plsc.subcore_barrier()

    pltpu.sync_copy(spg, locg)

    def scat(c, _):
        sl = pl.ds(c * 16, 16)
        vg = locg[pl.ds(0 * NS + c * 16, 16)]
        ag = locg[pl.ds(1 * NS + c * 16, 16)]
        fg = locg[pl.ds(2 * NS + c * 16, 16)]
        hg = locg[pl.ds(3 * NS + c * 16, 16)]
        is_now = (c % 8) < 4
        p = jnp.where(is_now, fg, hg)
        s = jnp.where(is_now, hg, fg)
        bvv = bv[sl]
        c8e = plsc.load_gather(cbuf, [bvv * 16])
        val = vg + c8e + ag * (p - vg + exv[sl] * s)
        iv = idxv[sl]
        m = (iv >= base) & (iv < base + CH)
        flat = bvv * CH + (iv - base)
        plsc.store_scatter(outbuf, [flat], val, mask=m)
        return 0

    lax.fori_loop(0, NS // 16, scat, 0)

    hs = [pltpu.async_copy(outbuf.at[pl.ds(b * CH, CH)],
                           out.at[pl.ds(b * VLEN + base, CH)], sem)
          for b in range(B)]
    for h in hs:
        h.wait()


def _pad_idx(x):
    return jnp.concatenate(
        [x, jnp.broadcast_to(x[:, -1:], (B, LP - L))], axis=1)


def kernel(company_emb, field_emb, nodes, com_id, hier_embed, raw_field_embed,
           raw_hier_embed, company_table, field_table, W_proj, b_proj, theta,
           alpha_fields, fc_field_w, fc_field_b, fc_company_w, fc_company_b,
           w1, b1, w2, b2):
    nodes = nodes.astype(jnp.int32)
    his = _pad_idx(nodes[:, 0, :])
    now = _pad_idx(nodes[:, 1, :])
    cid2 = com_id.astype(jnp.int32).reshape(1, B)

    thw = jnp.concatenate(
        [theta[:, 0],
         jnp.zeros((115 * 128 - NC,), jnp.float32)]).reshape(115, 128)
    aflat = jnp.concatenate(
        [alpha_fields[:, 0], jnp.zeros((VLEN - NF,), jnp.float32)])
    bproj2 = b_proj.reshape(1, DIM)
    fcb2 = fc_field_b.reshape(1, 1)
    fccb2 = fc_company_b.reshape(1, 1)
    b1r = b1.reshape(1, DIM // 2)
    b2r = b2.reshape(1, DIM)

    blk = lambda r, c: pl.BlockSpec((r, c), lambda i: (0, 0))
    vflat, fewflat, hmwflat, cb16, exm = pl.pallas_call(
        _dense_body,
        grid=(NB,),
        compiler_params=pltpu.CompilerParams(
            allow_input_fusion=[True, True, True] + [False] * 17),
        in_specs=[
            pl.BlockSpec((BLK, DIM), lambda i: (i, 0)),
            pl.BlockSpec((BLK, DIM), lambda i: (i, 0)),
            pl.BlockSpec((BLK, DIM), lambda i: (i, 0)),
            blk(DIM, DIM),
            blk(1, DIM),
            blk(1, DIM),
            blk(DIM, 1),
            blk(1, 1),
            blk(1, DIM),
            blk(1, 1),
            blk(B, DIM),
            blk(NC, DIM),
            blk(115, 128),
            pl.BlockSpec((1, B), lambda i: (0, 0),
                         memory_space=pltpu.SMEM),
            blk(B, LP),
            blk(B, LP),
            blk(DIM, DIM // 2),
            blk(1, DIM // 2),
            blk(DIM // 2, DIM),
            blk(1, DIM),
        ],
        out_specs=[
            pl.BlockSpec((BLK,), lambda i: (i,)),
            pl.BlockSpec((BLK,), lambda i: (i,)),
            pl.BlockSpec((BLK,), lambda i: (i,)),
            pl.BlockSpec((B, 16), lambda i: (0, 0)),
            pl.BlockSpec((B, 2 * LP), lambda i: (0, 0)),
        ],
        out_shape=[
            jax.ShapeDtypeStruct((VLEN,), jnp.float32),
            jax.ShapeDtypeStruct((VLEN,), jnp.float32),
            jax.ShapeDtypeStruct((VLEN,), jnp.float32),
            jax.ShapeDtypeStruct((B, 16), jnp.float32),
            jax.ShapeDtypeStruct((B, 2 * LP), jnp.float32),
        ],
        scratch_shapes=[
            pltpu.VMEM((B, DIM), jnp.float32),
            pltpu.VMEM((B, 1), jnp.float32),
        ],
    )(field_table.astype(jnp.bfloat16), field_emb.astype(jnp.bfloat16),
      raw_field_embed.astype(jnp.bfloat16), W_proj, bproj2, fc_field_w,
      fc_field_w.T, fcb2, fc_company_w, fccb2, company_emb, company_table,
      thw, cid2, now, his, w1.T, b1r, w2.T, b2r)

    scat_idx = jnp.concatenate([now, his], axis=1).reshape(NS)
    scat_b = jnp.broadcast_to(jnp.arange(B, dtype=jnp.int32)[:, None],
                              (B, 2 * LP)).reshape(NS)
    exflat = exm.reshape(NS)

    mesh = plsc.VectorSubcoreMesh(core_axis_name="c", subcore_axis_name="s")
    out = pl.kernel(
        _sc_body,
        out_type=jax.ShapeDtypeStruct((B * VLEN,), jnp.float32),
        mesh=mesh,
        compiler_params=pltpu.CompilerParams(needs_layout_passes=False),
        scratch_types=[
            pltpu.VMEM((VLEN,), jnp.float32),
            pltpu.VMEM((NS,), jnp.float32),
            pltpu.VMEM((B * CH,), jnp.float32),
            pltpu.VMEM((CH,), jnp.float32),
            pltpu.VMEM((B * 16,), jnp.float32),
            pltpu.VMEM((NS,), jnp.int32),
            pltpu.VMEM((NS,), jnp.int32),
            pltpu.VMEM((NS,), jnp.float32),
            pltpu.VMEM((4 * NS,), jnp.float32),
            pltpu.VMEM_SHARED((4 * NS,), jnp.float32),
            pltpu.SemaphoreType.DMA,
            pltpu.SemaphoreType.DMA,
        ],
    )(vflat, aflat, fewflat, hmwflat, cb16.reshape(B * 16), scat_idx,
      scat_b, exflat)

    return out.reshape(B, VLEN)[:, :NF]
